# Initial kernel scaffold; baseline (speedup 1.0000x reference)
import jax
import jax.numpy as jnp
from jax.experimental import pallas as pl


def kernel(x, edge_index, W1l, b1, W1r, W2l, b2, W2r):
    raise NotImplementedError("write your pallas kernel here")


# speedup vs baseline: 3.8807x; 3.8807x over previous
import functools

import jax
import jax.numpy as jnp
from jax import lax
from jax.experimental import pallas as pl
from jax.experimental.pallas import tpu as pltpu
from jax.experimental.pallas import tpu_sc as plsc

N_NODES = 10000
N_EDGES = 320000
D_FEAT = 128
HIDDEN = 128
N_CLASSES = 40
NC = 2
NS = 16
NW = NC * NS
CHUNK = 80
EDGES_PER_W = N_EDGES // NW
N_CHUNKS = EDGES_PER_W // CHUNK
N_PAD = 10240
ROWS_PER_SUB = N_PAD // NS
ZROWS = 32

_MESH = plsc.VectorSubcoreMesh(core_axis_name="c", subcore_axis_name="s")


def _zero_acc(zbuf, acc, width, row0):
    def zrow(i, _):
        for j in range(width // 16):
            zbuf[i, pl.ds(j * 16, 16)] = jnp.zeros((16,), jnp.float32)
        return 0
    lax.fori_loop(0, ZROWS, zrow, 0)
    for r in range(ROWS_PER_SUB // ZROWS):
        pltpu.sync_copy(zbuf, acc.at[pl.ds(row0 + r * ZROWS, ZROWS)])


@functools.partial(
    pl.kernel, mesh=_MESH,
    out_type=(jax.ShapeDtypeStruct((NC, N_PAD, D_FEAT), jnp.float32),),
    scratch_types=(
        pltpu.VMEM((CHUNK,), jnp.int32),
        pltpu.VMEM((CHUNK,), jnp.int32),
        pltpu.VMEM((CHUNK, D_FEAT), jnp.float32),
        pltpu.VMEM((ZROWS, D_FEAT), jnp.float32),
        pltpu.VMEM_SHARED((N_PAD, D_FEAT), jnp.float32),
        pltpu.SemaphoreType.DMA,
    ))
def _edge_agg(table, src, dst, out, src_v, dst_v, rows_v, zbuf, acc, sem):
    c = lax.axis_index("c")
    s = lax.axis_index("s")
    wid = s * NC + c
    row0 = s * ROWS_PER_SUB

    _zero_acc(zbuf, acc, D_FEAT, row0)
    plsc.subcore_barrier()

    base0 = wid * EDGES_PER_W

    def body(i, _):
        b = base0 + i * CHUNK
        pltpu.sync_copy(src.at[pl.ds(b, CHUNK)], src_v)
        pltpu.sync_copy(dst.at[pl.ds(b, CHUNK)], dst_v)
        pltpu.async_copy(table.at[src_v], rows_v, sem).wait()
        pltpu.sync_copy(rows_v, acc.at[dst_v], add=True)
        return 0
    lax.fori_loop(0, N_CHUNKS, body, 0)
    plsc.subcore_barrier()

    pltpu.sync_copy(acc.at[pl.ds(row0, ROWS_PER_SUB)],
                    out.at[c, pl.ds(row0, ROWS_PER_SUB)])


BS = 1000


def _tc1_body(agg_a, agg_b, deg_a, deg_b, x, w1l, b1, w1r,
              h_ref, dinv_ref):
    deg = jnp.maximum(deg_a[0, :, :1] + deg_b[0, :, :1], 1.0)
    dinv = 1.0 / deg
    agg = (agg_a[0] + agg_b[0]) * dinv
    h = jax.nn.relu(
        lax.dot_general(agg, w1l[...], (((1,), (1,)), ((), ())),
                        preferred_element_type=jnp.float32)
        + b1[...]
        + lax.dot_general(x[...], w1r[...], (((1,), (1,)), ((), ())),
                          preferred_element_type=jnp.float32))
    h_ref[...] = h
    dinv_ref[...] = jnp.broadcast_to(dinv, (BS, 16))


def _tc2_body(a2_a, a2_b, dinv, h, w2l, b2, w2r, out_ref):
    agg2 = (a2_a[0] + a2_b[0]) * dinv[:, :1]
    o = (lax.dot_general(agg2, w2l[...], (((1,), (1,)), ((), ())),
                         preferred_element_type=jnp.float32)
         + b2[...]
         + lax.dot_general(h[...], w2r[...], (((1,), (1,)), ((), ())),
                           preferred_element_type=jnp.float32))
    m = jnp.max(o, axis=1, keepdims=True)
    e = jnp.exp(o - m)
    out_ref[...] = o - m - jnp.log(jnp.sum(e, axis=1, keepdims=True))


def _row_spec(w):
    return pl.BlockSpec((BS, w), lambda i: (i, 0))


def _part_spec(w, j):
    return pl.BlockSpec((1, BS, w), lambda i, j=j: (j, i, 0))


def _full_spec(r, c):
    return pl.BlockSpec((r, c), lambda i: (0, 0))


def kernel(x, edge_index, W1l, b1, W1r, W2l, b2, W2r):
    src = edge_index[0]
    dst = edge_index[1]

    agg, = _edge_agg(x, src, dst)
    deg, = _edge_agg(jnp.ones((N_NODES, D_FEAT), jnp.float32), src, dst)

    grid = (N_NODES // BS,)
    h, dinv = pl.pallas_call(
        _tc1_body,
        grid=grid,
        in_specs=[_part_spec(D_FEAT, 0), _part_spec(D_FEAT, 1),
                  _part_spec(D_FEAT, 0), _part_spec(D_FEAT, 1),
                  _row_spec(D_FEAT),
                  _full_spec(HIDDEN, D_FEAT), _full_spec(1, HIDDEN),
                  _full_spec(HIDDEN, D_FEAT)],
        out_specs=[_row_spec(HIDDEN), _row_spec(16)],
        out_shape=[jax.ShapeDtypeStruct((N_NODES, HIDDEN), jnp.float32),
                   jax.ShapeDtypeStruct((N_NODES, 16), jnp.float32)],
    )(agg, agg, deg, deg, x, W1l, b1.reshape(1, HIDDEN), W1r)

    a2, = _edge_agg(h, src, dst)

    out = pl.pallas_call(
        _tc2_body,
        grid=grid,
        in_specs=[_part_spec(HIDDEN, 0), _part_spec(HIDDEN, 1), _row_spec(16),
                  _row_spec(HIDDEN), _full_spec(N_CLASSES, HIDDEN),
                  _full_spec(1, N_CLASSES), _full_spec(N_CLASSES, HIDDEN)],
        out_specs=_row_spec(N_CLASSES),
        out_shape=jax.ShapeDtypeStruct((N_NODES, N_CLASSES), jnp.float32),
    )(a2, a2, dinv, h, W2l, b2.reshape(1, N_CLASSES), W2r)
    return out


# speedup vs baseline: 4.8543x; 1.2509x over previous
import functools

import jax
import jax.numpy as jnp
from jax import lax
from jax.experimental import pallas as pl
from jax.experimental.pallas import tpu as pltpu
from jax.experimental.pallas import tpu_sc as plsc

N_NODES = 10000
N_EDGES = 320000
D_FEAT = 128
HIDDEN = 128
N_CLASSES = 40
NC = 2
NS = 16
NW = NC * NS
CHUNK = 128
EDGES_PER_W = N_EDGES // NW
N_CHUNKS = EDGES_PER_W // CHUNK
TAIL = EDGES_PER_W - N_CHUNKS * CHUNK
N_PAD = 10240
ROWS_PER_SUB = N_PAD // NS
ZROWS = 32

_MESH = plsc.VectorSubcoreMesh(core_axis_name="c", subcore_axis_name="s")


def _zero_acc(zbuf, acc, width, row0):
    def zrow(i, _):
        for j in range(width // 16):
            zbuf[i, pl.ds(j * 16, 16)] = jnp.zeros((16,), jnp.float32)
        return 0
    lax.fori_loop(0, ZROWS, zrow, 0)
    for r in range(ROWS_PER_SUB // ZROWS):
        pltpu.sync_copy(zbuf, acc.at[pl.ds(row0 + r * ZROWS, ZROWS)])


@functools.partial(
    pl.kernel, mesh=_MESH,
    out_type=(jax.ShapeDtypeStruct((NC, N_PAD, D_FEAT), jnp.float32),),
    scratch_types=(
        pltpu.VMEM((CHUNK,), jnp.int32),
        pltpu.VMEM((CHUNK,), jnp.int32),
        pltpu.VMEM((CHUNK, D_FEAT), jnp.float32),
        pltpu.VMEM((TAIL,), jnp.int32),
        pltpu.VMEM((TAIL,), jnp.int32),
        pltpu.VMEM((TAIL, D_FEAT), jnp.float32),
        pltpu.VMEM((ZROWS, D_FEAT), jnp.float32),
        pltpu.VMEM_SHARED((N_PAD, D_FEAT), jnp.float32),
        pltpu.SemaphoreType.DMA,
    ))
def _edge_agg(table, src, dst, out, src_v, dst_v, rows_v,
              srct_v, dstt_v, rowst_v, zbuf, acc, sem):
    c = lax.axis_index("c")
    s = lax.axis_index("s")
    wid = s * NC + c
    row0 = s * ROWS_PER_SUB

    _zero_acc(zbuf, acc, D_FEAT, row0)
    plsc.subcore_barrier()

    base0 = wid * EDGES_PER_W

    def body(i, _):
        b = base0 + i * CHUNK
        pltpu.sync_copy(src.at[pl.ds(b, CHUNK)], src_v)
        pltpu.sync_copy(dst.at[pl.ds(b, CHUNK)], dst_v)
        pltpu.async_copy(table.at[src_v], rows_v, sem).wait()
        pltpu.sync_copy(rows_v, acc.at[dst_v], add=True)
        return 0
    lax.fori_loop(0, N_CHUNKS, body, 0)

    bt = base0 + N_CHUNKS * CHUNK
    pltpu.sync_copy(src.at[pl.ds(bt, TAIL)], srct_v)
    pltpu.sync_copy(dst.at[pl.ds(bt, TAIL)], dstt_v)
    pltpu.async_copy(table.at[srct_v], rowst_v, sem).wait()
    pltpu.sync_copy(rowst_v, acc.at[dstt_v], add=True)
    plsc.subcore_barrier()

    pltpu.sync_copy(acc.at[pl.ds(row0, ROWS_PER_SUB)],
                    out.at[c, pl.ds(row0, ROWS_PER_SUB)])


BS = 1000


def _tc1_body(agg_a, agg_b, deg_a, deg_b, x, w1l, b1, w1r,
              h_ref, dinv_ref):
    deg = jnp.maximum(deg_a[0, :, :1] + deg_b[0, :, :1], 1.0)
    dinv = 1.0 / deg
    agg = (agg_a[0] + agg_b[0]) * dinv
    h = jax.nn.relu(
        lax.dot_general(agg, w1l[...], (((1,), (1,)), ((), ())),
                        preferred_element_type=jnp.float32)
        + b1[...]
        + lax.dot_general(x[...], w1r[...], (((1,), (1,)), ((), ())),
                          preferred_element_type=jnp.float32))
    h_ref[...] = h
    dinv_ref[...] = jnp.broadcast_to(dinv, (BS, 16))


def _tc2_body(a2_a, a2_b, dinv, h, w2l, b2, w2r, out_ref):
    agg2 = (a2_a[0] + a2_b[0]) * dinv[:, :1]
    o = (lax.dot_general(agg2, w2l[...], (((1,), (1,)), ((), ())),
                         preferred_element_type=jnp.float32)
         + b2[...]
         + lax.dot_general(h[...], w2r[...], (((1,), (1,)), ((), ())),
                           preferred_element_type=jnp.float32))
    m = jnp.max(o, axis=1, keepdims=True)
    e = jnp.exp(o - m)
    out_ref[...] = o - m - jnp.log(jnp.sum(e, axis=1, keepdims=True))


def _row_spec(w):
    return pl.BlockSpec((BS, w), lambda i: (i, 0))


def _part_spec(w, j):
    return pl.BlockSpec((1, BS, w), lambda i, j=j: (j, i, 0))


def _full_spec(r, c):
    return pl.BlockSpec((r, c), lambda i: (0, 0))


def kernel(x, edge_index, W1l, b1, W1r, W2l, b2, W2r):
    src = edge_index[0]
    dst = edge_index[1]

    agg, = _edge_agg(x, src, dst)
    deg, = _edge_agg(jnp.ones((N_NODES, D_FEAT), jnp.float32), src, dst)

    grid = (N_NODES // BS,)
    h, dinv = pl.pallas_call(
        _tc1_body,
        grid=grid,
        in_specs=[_part_spec(D_FEAT, 0), _part_spec(D_FEAT, 1),
                  _part_spec(D_FEAT, 0), _part_spec(D_FEAT, 1),
                  _row_spec(D_FEAT),
                  _full_spec(HIDDEN, D_FEAT), _full_spec(1, HIDDEN),
                  _full_spec(HIDDEN, D_FEAT)],
        out_specs=[_row_spec(HIDDEN), _row_spec(16)],
        out_shape=[jax.ShapeDtypeStruct((N_NODES, HIDDEN), jnp.float32),
                   jax.ShapeDtypeStruct((N_NODES, 16), jnp.float32)],
    )(agg, agg, deg, deg, x, W1l, b1.reshape(1, HIDDEN), W1r)

    a2, = _edge_agg(h, src, dst)

    out = pl.pallas_call(
        _tc2_body,
        grid=grid,
        in_specs=[_part_spec(HIDDEN, 0), _part_spec(HIDDEN, 1), _row_spec(16),
                  _row_spec(HIDDEN), _full_spec(N_CLASSES, HIDDEN),
                  _full_spec(1, N_CLASSES), _full_spec(N_CLASSES, HIDDEN)],
        out_specs=_row_spec(N_CLASSES),
        out_shape=jax.ShapeDtypeStruct((N_NODES, N_CLASSES), jnp.float32),
    )(a2, a2, dinv, h, W2l, b2.reshape(1, N_CLASSES), W2r)
    return out


# speedup vs baseline: 5.6160x; 1.1569x over previous
import functools

import jax
import jax.numpy as jnp
from jax import lax
from jax.experimental import pallas as pl
from jax.experimental.pallas import tpu as pltpu
from jax.experimental.pallas import tpu_sc as plsc

N_NODES = 10000
N_EDGES = 320000
D_FEAT = 128
HIDDEN = 128
N_CLASSES = 40
NC = 2
NS = 16
NW = NC * NS
CHUNK = 128
EDGES_PER_W = N_EDGES // NW
N_CHUNKS = EDGES_PER_W // CHUNK
TAIL = EDGES_PER_W - N_CHUNKS * CHUNK
N_PAD = 10240
ROWS_PER_SUB = N_PAD // NS
ZROWS = 32

_MESH = plsc.VectorSubcoreMesh(core_axis_name="c", subcore_axis_name="s")


def _zero_acc(zbuf, acc, width, row0):
    def zrow(i, _):
        for j in range(width // 16):
            zbuf[i, pl.ds(j * 16, 16)] = jnp.zeros((16,), jnp.float32)
        return 0
    lax.fori_loop(0, ZROWS, zrow, 0)
    for r in range(ROWS_PER_SUB // ZROWS):
        pltpu.sync_copy(zbuf, acc.at[pl.ds(row0 + r * ZROWS, ZROWS)])


@functools.partial(
    pl.kernel, mesh=_MESH,
    out_type=(jax.ShapeDtypeStruct((NC, N_PAD, D_FEAT), jnp.float32),),
    scratch_types=(
        pltpu.VMEM((CHUNK,), jnp.int32),
        pltpu.VMEM((CHUNK,), jnp.int32),
        pltpu.VMEM((CHUNK, D_FEAT), jnp.float32),
        pltpu.VMEM((TAIL,), jnp.int32),
        pltpu.VMEM((TAIL,), jnp.int32),
        pltpu.VMEM((TAIL, D_FEAT), jnp.float32),
        pltpu.VMEM((ZROWS, D_FEAT), jnp.float32),
        pltpu.VMEM_SHARED((N_PAD, D_FEAT), jnp.float32),
        pltpu.SemaphoreType.DMA,
    ))
def _edge_agg(table, src, dst, out, src_v, dst_v, rows_v,
              srct_v, dstt_v, rowst_v, zbuf, acc, sem):
    c = lax.axis_index("c")
    s = lax.axis_index("s")
    wid = s * NC + c
    row0 = s * ROWS_PER_SUB

    _zero_acc(zbuf, acc, D_FEAT, row0)
    plsc.subcore_barrier()

    base0 = wid * EDGES_PER_W

    def body(i, _):
        b = base0 + i * CHUNK
        pltpu.sync_copy(src.at[pl.ds(b, CHUNK)], src_v)
        cp = pltpu.async_copy(table.at[src_v], rows_v, sem)
        pltpu.sync_copy(dst.at[pl.ds(b, CHUNK)], dst_v)
        cp.wait()
        pltpu.sync_copy(rows_v, acc.at[dst_v], add=True)
        return 0
    lax.fori_loop(0, N_CHUNKS, body, 0)

    bt = base0 + N_CHUNKS * CHUNK
    pltpu.sync_copy(src.at[pl.ds(bt, TAIL)], srct_v)
    pltpu.sync_copy(dst.at[pl.ds(bt, TAIL)], dstt_v)
    pltpu.async_copy(table.at[srct_v], rowst_v, sem).wait()
    pltpu.sync_copy(rowst_v, acc.at[dstt_v], add=True)
    plsc.subcore_barrier()

    pltpu.sync_copy(acc.at[pl.ds(row0, ROWS_PER_SUB)],
                    out.at[c, pl.ds(row0, ROWS_PER_SUB)])


BS = 1000


def _tc1_body(agg_a, agg_b, deg_a, deg_b, x, w1l, b1, w1r,
              h_ref, dinv_ref):
    deg = jnp.maximum(deg_a[0, :, :1] + deg_b[0, :, :1], 1.0)
    dinv = 1.0 / deg
    agg = (agg_a[0] + agg_b[0]) * dinv
    h = jax.nn.relu(
        lax.dot_general(agg, w1l[...], (((1,), (1,)), ((), ())),
                        preferred_element_type=jnp.float32)
        + b1[...]
        + lax.dot_general(x[...], w1r[...], (((1,), (1,)), ((), ())),
                          preferred_element_type=jnp.float32))
    h_ref[...] = h
    dinv_ref[...] = jnp.broadcast_to(dinv, (BS, 16))


def _tc2_body(a2_a, a2_b, dinv, h, w2l, b2, w2r, out_ref):
    agg2 = (a2_a[0] + a2_b[0]) * dinv[:, :1]
    o = (lax.dot_general(agg2, w2l[...], (((1,), (1,)), ((), ())),
                         preferred_element_type=jnp.float32)
         + b2[...]
         + lax.dot_general(h[...], w2r[...], (((1,), (1,)), ((), ())),
                           preferred_element_type=jnp.float32))
    m = jnp.max(o, axis=1, keepdims=True)
    e = jnp.exp(o - m)
    out_ref[...] = o - m - jnp.log(jnp.sum(e, axis=1, keepdims=True))


def _row_spec(w):
    return pl.BlockSpec((BS, w), lambda i: (i, 0))


def _part_spec(w, j):
    return pl.BlockSpec((1, BS, w), lambda i, j=j: (j, i, 0))


def _full_spec(r, c):
    return pl.BlockSpec((r, c), lambda i: (0, 0))


def kernel(x, edge_index, W1l, b1, W1r, W2l, b2, W2r):
    src = edge_index[0]
    dst = edge_index[1]

    agg, = _edge_agg(x, src, dst)
    deg, = _edge_agg(jnp.ones((N_NODES, D_FEAT), jnp.float32), src, dst)

    grid = (N_NODES // BS,)
    h, dinv = pl.pallas_call(
        _tc1_body,
        grid=grid,
        in_specs=[_part_spec(D_FEAT, 0), _part_spec(D_FEAT, 1),
                  _part_spec(D_FEAT, 0), _part_spec(D_FEAT, 1),
                  _row_spec(D_FEAT),
                  _full_spec(HIDDEN, D_FEAT), _full_spec(1, HIDDEN),
                  _full_spec(HIDDEN, D_FEAT)],
        out_specs=[_row_spec(HIDDEN), _row_spec(16)],
        out_shape=[jax.ShapeDtypeStruct((N_NODES, HIDDEN), jnp.float32),
                   jax.ShapeDtypeStruct((N_NODES, 16), jnp.float32)],
    )(agg, agg, deg, deg, x, W1l, b1.reshape(1, HIDDEN), W1r)

    a2, = _edge_agg(h, src, dst)

    out = pl.pallas_call(
        _tc2_body,
        grid=grid,
        in_specs=[_part_spec(HIDDEN, 0), _part_spec(HIDDEN, 1), _row_spec(16),
                  _row_spec(HIDDEN), _full_spec(N_CLASSES, HIDDEN),
                  _full_spec(1, N_CLASSES), _full_spec(N_CLASSES, HIDDEN)],
        out_specs=_row_spec(N_CLASSES),
        out_shape=jax.ShapeDtypeStruct((N_NODES, N_CLASSES), jnp.float32),
    )(a2, a2, dinv, h, W2l, b2.reshape(1, N_CLASSES), W2r)
    return out


# speedup vs baseline: 7.2755x; 1.2955x over previous
import functools

import jax
import jax.numpy as jnp
from jax import lax
from jax.experimental import pallas as pl
from jax.experimental.pallas import tpu as pltpu
from jax.experimental.pallas import tpu_sc as plsc

N_NODES = 10000
N_EDGES = 320000
D_FEAT = 128
HIDDEN = 128
N_CLASSES = 40
NC = 2
NS = 16
NW = NC * NS
CHUNK = 128
EDGES_PER_W = N_EDGES // NW
N_CHUNKS = EDGES_PER_W // CHUNK
TAIL = EDGES_PER_W - N_CHUNKS * CHUNK
N_PAD = 10240
ROWS_PER_SUB = N_PAD // NS
ZROWS = 32

_MESH = plsc.VectorSubcoreMesh(core_axis_name="c", subcore_axis_name="s")


def _zero_acc(zbuf, acc, width, row0):
    def zrow(i, _):
        for j in range(width // 16):
            zbuf[i, pl.ds(j * 16, 16)] = jnp.zeros((16,), jnp.float32)
        return 0
    lax.fori_loop(0, ZROWS, zrow, 0)
    for r in range(ROWS_PER_SUB // ZROWS):
        pltpu.sync_copy(zbuf, acc.at[pl.ds(row0 + r * ZROWS, ZROWS)])


@functools.partial(
    pl.kernel, mesh=_MESH,
    out_type=(jax.ShapeDtypeStruct((NC, N_PAD, D_FEAT), jnp.float32),),
    scratch_types=(
        pltpu.VMEM((CHUNK,), jnp.int32),
        pltpu.VMEM((CHUNK,), jnp.int32),
        pltpu.VMEM((CHUNK, D_FEAT), jnp.float32),
        pltpu.VMEM((CHUNK,), jnp.int32),
        pltpu.VMEM((CHUNK,), jnp.int32),
        pltpu.VMEM((CHUNK, D_FEAT), jnp.float32),
        pltpu.VMEM((TAIL,), jnp.int32),
        pltpu.VMEM((TAIL,), jnp.int32),
        pltpu.VMEM((TAIL, D_FEAT), jnp.float32),
        pltpu.VMEM((ZROWS, D_FEAT), jnp.float32),
        pltpu.VMEM_SHARED((N_PAD, D_FEAT), jnp.float32),
        pltpu.SemaphoreType.DMA,
        pltpu.SemaphoreType.DMA,
    ))
def _edge_agg(table, src, dst, out, src_a, dst_a, rows_a,
              src_b, dst_b, rows_b, srct_v, dstt_v, rowst_v, zbuf, acc,
              sem_a, sem_b):
    c = lax.axis_index("c")
    s = lax.axis_index("s")
    wid = s * NC + c
    row0 = s * ROWS_PER_SUB

    _zero_acc(zbuf, acc, D_FEAT, row0)
    plsc.subcore_barrier()

    base0 = wid * EDGES_PER_W

    def body(j, _):
        b0 = base0 + (2 * j) * CHUNK
        b1 = b0 + CHUNK
        pltpu.sync_copy(src.at[pl.ds(b0, CHUNK)], src_a)
        cpa = pltpu.async_copy(table.at[src_a], rows_a, sem_a)
        pltpu.sync_copy(dst.at[pl.ds(b0, CHUNK)], dst_a)
        pltpu.sync_copy(src.at[pl.ds(b1, CHUNK)], src_b)
        cpb = pltpu.async_copy(table.at[src_b], rows_b, sem_b)
        pltpu.sync_copy(dst.at[pl.ds(b1, CHUNK)], dst_b)
        cpa.wait()
        pltpu.sync_copy(rows_a, acc.at[dst_a], add=True)
        cpb.wait()
        pltpu.sync_copy(rows_b, acc.at[dst_b], add=True)
        return 0
    lax.fori_loop(0, N_CHUNKS // 2, body, 0)

    bt = base0 + N_CHUNKS * CHUNK
    pltpu.sync_copy(src.at[pl.ds(bt, TAIL)], srct_v)
    pltpu.sync_copy(dst.at[pl.ds(bt, TAIL)], dstt_v)
    pltpu.async_copy(table.at[srct_v], rowst_v, sem_a).wait()
    pltpu.sync_copy(rowst_v, acc.at[dstt_v], add=True)
    plsc.subcore_barrier()

    pltpu.sync_copy(acc.at[pl.ds(row0, ROWS_PER_SUB)],
                    out.at[c, pl.ds(row0, ROWS_PER_SUB)])


BS = 1000


def _tc1_body(agg_a, agg_b, deg_a, deg_b, x, w1l, b1, w1r,
              h_ref, dinv_ref):
    deg = jnp.maximum(deg_a[0, :, :1] + deg_b[0, :, :1], 1.0)
    dinv = 1.0 / deg
    agg = (agg_a[0] + agg_b[0]) * dinv
    h = jax.nn.relu(
        lax.dot_general(agg, w1l[...], (((1,), (1,)), ((), ())),
                        preferred_element_type=jnp.float32)
        + b1[...]
        + lax.dot_general(x[...], w1r[...], (((1,), (1,)), ((), ())),
                          preferred_element_type=jnp.float32))
    h_ref[...] = h
    dinv_ref[...] = jnp.broadcast_to(dinv, (BS, 16))


def _tc2_body(a2_a, a2_b, dinv, h, w2l, b2, w2r, out_ref):
    agg2 = (a2_a[0] + a2_b[0]) * dinv[:, :1]
    o = (lax.dot_general(agg2, w2l[...], (((1,), (1,)), ((), ())),
                         preferred_element_type=jnp.float32)
         + b2[...]
         + lax.dot_general(h[...], w2r[...], (((1,), (1,)), ((), ())),
                           preferred_element_type=jnp.float32))
    m = jnp.max(o, axis=1, keepdims=True)
    e = jnp.exp(o - m)
    out_ref[...] = o - m - jnp.log(jnp.sum(e, axis=1, keepdims=True))


def _row_spec(w):
    return pl.BlockSpec((BS, w), lambda i: (i, 0))


def _part_spec(w, j):
    return pl.BlockSpec((1, BS, w), lambda i, j=j: (j, i, 0))


def _full_spec(r, c):
    return pl.BlockSpec((r, c), lambda i: (0, 0))


def kernel(x, edge_index, W1l, b1, W1r, W2l, b2, W2r):
    src = edge_index[0]
    dst = edge_index[1]

    agg, = _edge_agg(x, src, dst)
    deg, = _edge_agg(jnp.ones((N_NODES, D_FEAT), jnp.float32), src, dst)

    grid = (N_NODES // BS,)
    h, dinv = pl.pallas_call(
        _tc1_body,
        grid=grid,
        in_specs=[_part_spec(D_FEAT, 0), _part_spec(D_FEAT, 1),
                  _part_spec(D_FEAT, 0), _part_spec(D_FEAT, 1),
                  _row_spec(D_FEAT),
                  _full_spec(HIDDEN, D_FEAT), _full_spec(1, HIDDEN),
                  _full_spec(HIDDEN, D_FEAT)],
        out_specs=[_row_spec(HIDDEN), _row_spec(16)],
        out_shape=[jax.ShapeDtypeStruct((N_NODES, HIDDEN), jnp.float32),
                   jax.ShapeDtypeStruct((N_NODES, 16), jnp.float32)],
    )(agg, agg, deg, deg, x, W1l, b1.reshape(1, HIDDEN), W1r)

    a2, = _edge_agg(h, src, dst)

    out = pl.pallas_call(
        _tc2_body,
        grid=grid,
        in_specs=[_part_spec(HIDDEN, 0), _part_spec(HIDDEN, 1), _row_spec(16),
                  _row_spec(HIDDEN), _full_spec(N_CLASSES, HIDDEN),
                  _full_spec(1, N_CLASSES), _full_spec(N_CLASSES, HIDDEN)],
        out_specs=_row_spec(N_CLASSES),
        out_shape=jax.ShapeDtypeStruct((N_NODES, N_CLASSES), jnp.float32),
    )(a2, a2, dinv, h, W2l, b2.reshape(1, N_CLASSES), W2r)
    return out
